# baseline (speedup 1.0000x reference)
import functools

import jax
import jax.numpy as jnp
from jax import lax
from jax.experimental import pallas as pl
from jax.experimental.pallas import tpu as pltpu
from jax.experimental.pallas import tpu_sc as plsc

B = 16384
L = 50
EMB = 32
C = 3

NC = 2
NS = 16
NW = NC * NS
RPW = B // NW
NWAVE = 16
WB = RPW // NWAVE
WROWS = WB * L
FPW = RPW * L
NA = 896


def _sc_partial(x1, table, wt):
  mesh = plsc.VectorSubcoreMesh(core_axis_name="c", subcore_axis_name="s")

  @functools.partial(
      pl.kernel,
      mesh=mesh,
      compiler_params=pltpu.CompilerParams(use_tc_tiling_on_sc=False),
      out_type=jax.ShapeDtypeStruct((B, 128), jnp.float32),
      scratch_types=[
          pltpu.VMEM((2, WROWS), jnp.int32),
          pltpu.VMEM((WROWS, EMB), jnp.float32),
          pltpu.VMEM((L, 128), jnp.float32),
          pltpu.VMEM((WB, 128), jnp.float32),
          pltpu.SemaphoreType.DMA,
          pltpu.SemaphoreType.DMA,
          pltpu.SemaphoreType.DMA,
      ],
  )
  def k(x1_hbm, table_hbm, wt_hbm, p_hbm, idx_v, rows, wt_v, out_v,
        sem_a, sem_b, sem_i):
    wid = lax.axis_index("s") * NC + lax.axis_index("c")
    row0 = wid * RPW
    flat0 = wid * FPW
    pltpu.sync_copy(wt_hbm, wt_v)

    zero = jnp.zeros((16,), jnp.float32)
    for r in range(WB):
      for h in range(C, 8):
        out_v[r, pl.ds(h * 16, 16)] = zero

    def fire_wave(q):
      for g in range(13):
        n0 = min(g * 128, WROWS)
        n1 = min((g + 1) * 128, WROWS)
        pltpu.async_copy(
            table_hbm.at[idx_v.at[q, pl.ds(n0, n1 - n0)]],
            rows.at[pl.ds(n0, n1 - n0)],
            sem_a if n1 <= NA else sem_b,
        )

    def drain(n0, n1, sem):
      pltpu.make_async_copy(
          table_hbm.at[pl.ds(0, n1 - n0)], rows.at[pl.ds(n0, n1 - n0)], sem
      ).wait()

    pltpu.sync_copy(x1_hbm.at[pl.ds(flat0, WROWS)], idx_v.at[0])
    fire_wave(0)
    pltpu.async_copy(
        x1_hbm.at[pl.ds(flat0 + WROWS, WROWS)], idx_v.at[1], sem_i)

    def wave_body(i, _):
      q = lax.rem(i, 2)

      def compute16(hw, _):
        @pl.when(hw == 0)
        def _():
          drain(0, NA, sem_a)

        @pl.when(hw == 1)
        def _():
          drain(NA, WROWS, sem_b)

        def cbody(sc8, _):
          base = hw * 800 + sc8 * (8 * L)
          acc = [zero] * (C * 8)
          for l in range(L):
            w = [(wt_v[l, pl.ds(c * 32, 16)],
                  wt_v[l, pl.ds(c * 32 + 16, 16)]) for c in range(C)]
            for j in range(8):
              f = base + (j * L + l)
              r0 = rows[f, 0:16]
              r1 = rows[f, 16:32]
              for c in range(C):
                acc[c * 8 + j] = acc[c * 8 + j] + r0 * w[c][0] + r1 * w[c][1]
          row16 = hw * 16 + sc8 * 8
          for j in range(8):
            for c in range(C):
              out_v[row16 + j, pl.ds(c * 16, 16)] = acc[c * 8 + j]
          return ()

        lax.fori_loop(0, 2, cbody, ())
        return ()

      lax.fori_loop(0, 2, compute16, ())

      @pl.when(i + 1 < NWAVE)
      def _():
        pltpu.make_async_copy(
            x1_hbm.at[pl.ds(0, WROWS)], idx_v.at[1 - q], sem_i).wait()
        fire_wave(1 - q)

      @pl.when(i + 2 < NWAVE)
      def _():
        pltpu.async_copy(
            x1_hbm.at[pl.ds(flat0 + (i + 2) * WROWS, WROWS)],
            idx_v.at[q], sem_i)

      pltpu.sync_copy(out_v, p_hbm.at[pl.ds(row0 + i * WB, WB)])
      return ()

    lax.fori_loop(0, NWAVE, wave_body, ())

  return k(x1, table, wt)


def _tc_finish(p, bias):
  blk = 2048

  def body(p_ref, b_ref, o_ref):
    z = [
        jnp.sum(p_ref[:, c * 16:(c + 1) * 16], axis=-1, keepdims=True)
        + b_ref[0, c]
        for c in range(C)
    ]
    m = jnp.maximum(jnp.maximum(z[0], z[1]), z[2])
    s = jnp.exp(z[0] - m) + jnp.exp(z[1] - m) + jnp.exp(z[2] - m)
    ln = jnp.log(s)
    for c in range(C):
      o_ref[:, c:c + 1] = z[c] - m - ln

  return pl.pallas_call(
      body,
      grid=(B // blk,),
      in_specs=[
          pl.BlockSpec((blk, 128), lambda i: (i, 0)),
          pl.BlockSpec((1, C), lambda i: (0, 0)),
      ],
      out_specs=pl.BlockSpec((blk, C), lambda i: (i, 0)),
      out_shape=jax.ShapeDtypeStruct((B, C), jnp.float32),
  )(p, bias.reshape(1, C))


def kernel(x, table, W, b):
  x1 = x.astype(jnp.int32).reshape(B * L)
  wt = jnp.pad(
      W.reshape(C, L, EMB).transpose(1, 0, 2).reshape(L, C * EMB),
      ((0, 0), (0, 128 - C * EMB)),
  )
  p = _sc_partial(x1, table, wt)
  return _tc_finish(p, b)


# speedup vs baseline: 1.1478x; 1.1478x over previous
import functools

import jax
import jax.numpy as jnp
from jax import lax
from jax.experimental import pallas as pl
from jax.experimental.pallas import tpu as pltpu
from jax.experimental.pallas import tpu_sc as plsc

B = 16384
L = 50
EMB = 32
C = 3

NC = 2
NS = 16
NW = NC * NS
RPW = B // NW
NWAVE = 16
WB = RPW // NWAVE
WROWS = WB * L
FPW = RPW * L
NA = 896


def _sc_partial(x1, table, wt):
  mesh = plsc.VectorSubcoreMesh(core_axis_name="c", subcore_axis_name="s")

  @functools.partial(
      pl.kernel,
      mesh=mesh,
      compiler_params=pltpu.CompilerParams(use_tc_tiling_on_sc=False),
      out_type=jax.ShapeDtypeStruct((B, 128), jnp.float32),
      scratch_types=[
          pltpu.VMEM((2, WROWS), jnp.int32),
          pltpu.VMEM((WROWS, EMB), jnp.float32),
          pltpu.VMEM((L, 128), jnp.float32),
          pltpu.VMEM((WB, 128), jnp.float32),
          pltpu.SemaphoreType.DMA,
          pltpu.SemaphoreType.DMA,
          pltpu.SemaphoreType.DMA,
      ],
  )
  def k(x1_hbm, table_hbm, wt_hbm, p_hbm, idx_v, rows, wt_v, out_v,
        sem_a, sem_b, sem_i):
    wid = lax.axis_index("s") * NC + lax.axis_index("c")
    row0 = wid * RPW
    flat0 = wid * FPW
    pltpu.sync_copy(wt_hbm, wt_v)

    zero = jnp.zeros((16,), jnp.float32)
    for r in range(WB):
      for h in range(C, 8):
        out_v[r, pl.ds(h * 16, 16)] = zero

    def fire_wave(q):
      for g in range(13):
        n0 = min(g * 128, WROWS)
        n1 = min((g + 1) * 128, WROWS)
        pltpu.async_copy(
            table_hbm.at[idx_v.at[q, pl.ds(n0, n1 - n0)]],
            rows.at[pl.ds(n0, n1 - n0)],
            sem_a if n1 <= NA else sem_b,
        )

    def drain(n0, n1, sem):
      pltpu.make_async_copy(
          table_hbm.at[pl.ds(0, n1 - n0)], rows.at[pl.ds(n0, n1 - n0)], sem
      ).wait()

    pltpu.sync_copy(x1_hbm.at[pl.ds(flat0, WROWS)], idx_v.at[0])
    fire_wave(0)
    pltpu.async_copy(
        x1_hbm.at[pl.ds(flat0 + WROWS, WROWS)], idx_v.at[1], sem_i)

    def wave_body(i, _):
      q = lax.rem(i, 2)

      def compute16(hw, _):
        @pl.when(hw == 0)
        def _():
          drain(0, NA, sem_a)

        @pl.when(hw == 1)
        def _():
          drain(NA, WROWS, sem_b)

        def cbody(sc4, _):
          base = hw * 800 + sc4 * (4 * L)
          acc = [zero] * (C * 4)
          for l in range(L):
            w = [(wt_v[l, pl.ds(c * 32, 16)],
                  wt_v[l, pl.ds(c * 32 + 16, 16)]) for c in range(C)]
            for j in range(4):
              f = base + (j * L + l)
              r0 = rows[f, 0:16]
              r1 = rows[f, 16:32]
              for c in range(C):
                acc[c * 4 + j] = acc[c * 4 + j] + r0 * w[c][0] + r1 * w[c][1]
          row16 = hw * 16 + sc4 * 4
          for j in range(4):
            for c in range(C):
              out_v[row16 + j, pl.ds(c * 16, 16)] = acc[c * 4 + j]
          return ()

        lax.fori_loop(0, 4, cbody, ())
        return ()

      lax.fori_loop(0, 2, compute16, ())

      @pl.when(i + 1 < NWAVE)
      def _():
        pltpu.make_async_copy(
            x1_hbm.at[pl.ds(0, WROWS)], idx_v.at[1 - q], sem_i).wait()
        fire_wave(1 - q)

      @pl.when(i + 2 < NWAVE)
      def _():
        pltpu.async_copy(
            x1_hbm.at[pl.ds(flat0 + (i + 2) * WROWS, WROWS)],
            idx_v.at[q], sem_i)

      pltpu.sync_copy(out_v, p_hbm.at[pl.ds(row0 + i * WB, WB)])
      return ()

    lax.fori_loop(0, NWAVE, wave_body, ())

  return k(x1, table, wt)


def _tc_finish(p, bias):
  blk = 2048

  def body(p_ref, b_ref, o_ref):
    z = [
        jnp.sum(p_ref[:, c * 16:(c + 1) * 16], axis=-1, keepdims=True)
        + b_ref[0, c]
        for c in range(C)
    ]
    m = jnp.maximum(jnp.maximum(z[0], z[1]), z[2])
    s = jnp.exp(z[0] - m) + jnp.exp(z[1] - m) + jnp.exp(z[2] - m)
    ln = jnp.log(s)
    for c in range(C):
      o_ref[:, c:c + 1] = z[c] - m - ln

  return pl.pallas_call(
      body,
      grid=(B // blk,),
      in_specs=[
          pl.BlockSpec((blk, 128), lambda i: (i, 0)),
          pl.BlockSpec((1, C), lambda i: (0, 0)),
      ],
      out_specs=pl.BlockSpec((blk, C), lambda i: (i, 0)),
      out_shape=jax.ShapeDtypeStruct((B, C), jnp.float32),
  )(p, bias.reshape(1, C))


def kernel(x, table, W, b):
  x1 = x.astype(jnp.int32).reshape(B * L)
  wt = jnp.pad(
      W.reshape(C, L, EMB).transpose(1, 0, 2).reshape(L, C * EMB),
      ((0, 0), (0, 128 - C * EMB)),
  )
  p = _sc_partial(x1, table, wt)
  return _tc_finish(p, b)
